# baseline (speedup 1.0000x reference)
import functools

import jax
import jax.numpy as jnp
from jax import lax
from jax.experimental import pallas as pl
from jax.experimental.pallas import tpu as pltpu
from jax.experimental.pallas import tpu_sc as plsc

_LANES = 16


def _sc_embed(idx_flat, tok_emb, pos_emb, t_period):
    n = idx_flat.shape[0]
    d = tok_emb.shape[1]
    nw = 32
    per_w = n // nw
    ch = t_period
    n_chunks = per_w // ch
    d_slices = d // _LANES

    mesh = plsc.VectorSubcoreMesh(core_axis_name="c", subcore_axis_name="s")

    @functools.partial(
        pl.kernel,
        out_type=jax.ShapeDtypeStruct((n, d), jnp.float32),
        mesh=mesh,
        compiler_params=pltpu.CompilerParams(use_tc_tiling_on_sc=False),
        scratch_types=[
            pltpu.VMEM((ch,), jnp.int32),
            pltpu.VMEM((ch, d), jnp.float32),
            pltpu.VMEM((ch, d), jnp.float32),
            pltpu.SemaphoreType.DMA,
        ],
    )
    def k(idx_hbm, tok_hbm, pos_hbm, out_hbm, idx_v, buf, pospat, sem):
        wid = lax.axis_index("s") * 2 + lax.axis_index("c")
        base = wid * per_w
        pltpu.sync_copy(pos_hbm.at[pl.ds(0, ch)], pospat)

        def chunk_body(u, carry):
            row0 = base + u * ch
            pltpu.sync_copy(idx_hbm.at[pl.ds(row0, ch)], idx_v)
            pltpu.async_copy(tok_hbm.at[idx_v], buf, sem).wait()

            def add_row(j, c2):
                for s in range(d_slices):
                    sl = pl.ds(s * _LANES, _LANES)
                    buf[j, sl] = buf[j, sl] + pospat[j, sl]
                return c2

            lax.fori_loop(0, ch, add_row, 0)
            pltpu.sync_copy(buf, out_hbm.at[pl.ds(row0, ch)])
            return carry

        lax.fori_loop(0, n_chunks, chunk_body, 0)

    return k(idx_flat, tok_emb, pos_emb)


def kernel(idx, tok_emb, pos_emb):
    b, t = idx.shape
    d = tok_emb.shape[1]
    flat = idx.reshape(b * t).astype(jnp.int32)
    out = _sc_embed(flat, tok_emb, pos_emb, t)
    return out.reshape(b, t, d)
